# baseline (speedup 1.0000x reference)
import functools

import jax
import jax.numpy as jnp
from jax import lax
from jax.experimental import pallas as pl
from jax.experimental.pallas import tpu as pltpu
from jax.experimental.pallas import tpu_sc as plsc

N = 10000
E = 320000
R = 8
NC, NS = 2, 16
NW = NC * NS
W = 80
NROWS = E // W
RPT = NROWS // NW
RPT_CNT = NROWS // NS
CNT_PAD = 81920
L = 16


def _vsl(j):
    return (j // 5, pl.ds((j % 5) * L, L))


def _sc_counts_norm(src, dst, etype):
    mesh = plsc.VectorSubcoreMesh(core_axis_name="c", subcore_axis_name="s")
    CH = 2000
    CR = CH // W

    @functools.partial(
        pl.kernel,
        out_type=[
            jax.ShapeDtypeStruct((NROWS, W), jnp.int32),
            jax.ShapeDtypeStruct((NROWS, W), jnp.float32),
            jax.ShapeDtypeStruct((NROWS, W), jnp.int32),
        ],
        mesh=mesh,
        compiler_params=pltpu.CompilerParams(use_tc_tiling_on_sc=False, needs_layout_passes=False),
        scratch_types=[
            pltpu.VMEM_SHARED((CNT_PAD,), jnp.float32),
            pltpu.VMEM((CH,), jnp.int32),
            pltpu.VMEM((CH,), jnp.int32),
            pltpu.VMEM((CH,), jnp.int32),
            pltpu.VMEM((CR, W), jnp.int32),
            pltpu.VMEM((CR, W), jnp.int32),
            pltpu.VMEM((CR, W), jnp.int32),
            pltpu.VMEM((CR, W), jnp.float32),
            pltpu.VMEM((W,), jnp.float32),
            pltpu.VMEM((1024,), jnp.float32),
            pltpu.VMEM((CNT_PAD,), jnp.float32),
        ],
    )
    def k(src_h, dst_h, type_h, gidx_h, norm_h, dstr_h,
          counts_sh, sbuf, dbuf, tbuf, cidx, gbuf, drows, nrows_, ones, zbuf,
          cloc):
        cid = lax.axis_index("c")
        sid = lax.axis_index("s")
        wid = sid * NC + cid

        def initz(i, _):
            zbuf[pl.ds(i * L, L)] = jnp.zeros((L,), jnp.float32)
            return _
        lax.fori_loop(0, 1024 // L, initz, None)
        for j in range(W // L):
            ones[pl.ds(j * L, L)] = jnp.ones((L,), jnp.float32)
        def zrow(i, _):
            pltpu.sync_copy(zbuf, counts_sh.at[pl.ds(sid * 5120 + i * 1024, 1024)])
            return _
        lax.fori_loop(0, 5, zrow, None)
        plsc.subcore_barrier()

        def count_chunk(c, _):
            base = sid * (RPT_CNT * W) + c * CH
            pltpu.sync_copy(dst_h.at[pl.ds(base, CH)], dbuf)
            pltpu.sync_copy(type_h.at[pl.ds(base, CH)], tbuf)

            def mkcidx(r, _):
                for j in range(W // L):
                    s = r * W + j * L
                    d = dbuf[pl.ds(s, L)]
                    t = tbuf[pl.ds(s, L)]
                    cidx[r, pl.ds(j * L, L)] = d * R + t
                return _
            lax.fori_loop(0, CR, mkcidx, None)

            def scat(r, _):
                pltpu.sync_copy(ones, counts_sh.at[cidx.at[r]], add=True)
                return _
            lax.fori_loop(0, CR, scat, None)
            return _
        lax.fori_loop(0, RPT_CNT * W // CH, count_chunk, None)
        plsc.subcore_barrier()

        pltpu.sync_copy(counts_sh, cloc)

        def norm_chunk(c, _):
            base = wid * (RPT * W) + c * CH
            row0 = wid * RPT + c * CR
            pltpu.sync_copy(src_h.at[pl.ds(base, CH)], sbuf)
            pltpu.sync_copy(dst_h.at[pl.ds(base, CH)], dbuf)
            pltpu.sync_copy(type_h.at[pl.ds(base, CH)], tbuf)

            def mkrow(r, _):
                for j in range(W // L):
                    s = r * W + j * L
                    sj = pl.ds(j * L, L)
                    d = dbuf[pl.ds(s, L)]
                    t = tbuf[pl.ds(s, L)]
                    sc = sbuf[pl.ds(s, L)]
                    gbuf[r, sj] = t * N + sc
                    drows[r, sj] = d
                    cnt = plsc.load_gather(cloc, [d * R + t])
                    nrows_[r, sj] = 1.0 / jnp.maximum(cnt, 1.0)
                return _
            lax.fori_loop(0, CR, mkrow, None)

            pltpu.sync_copy(gbuf, gidx_h.at[pl.ds(row0, CR)])
            pltpu.sync_copy(nrows_, norm_h.at[pl.ds(row0, CR)])
            pltpu.sync_copy(drows, dstr_h.at[pl.ds(row0, CR)])
            return _
        lax.fori_loop(0, RPT * W // CH, norm_chunk, None)

    return k(src, dst, etype)


def _sc_msgpass(y_flat, gidx, norm, dstr, D):
    mesh = plsc.VectorSubcoreMesh(core_axis_name="c", subcore_axis_name="s")
    GR = 5
    NG = RPT // GR
    NB = 3

    @functools.partial(
        pl.kernel,
        out_type=jax.ShapeDtypeStruct((NC, N, D), jnp.float32),
        mesh=mesh,
        compiler_params=pltpu.CompilerParams(use_tc_tiling_on_sc=False, needs_layout_passes=False),
        scratch_types=(
            [pltpu.VMEM_SHARED((N, D), jnp.float32)]
            + [pltpu.VMEM((GR, W, D), jnp.float32) for _ in range(NB)]
            + [pltpu.VMEM((GR, W), jnp.int32) for _ in range(NB)]
            + [pltpu.VMEM((GR, W), jnp.float32) for _ in range(NB)]
            + [pltpu.VMEM((GR, W), jnp.int32) for _ in range(NB)]
            + [pltpu.VMEM((25, D), jnp.float32)]
            + [pltpu.SemaphoreType.DMA for _ in range(2 * NB)]
        ),
    )
    def k(y_h, gidx_h, norm_h, dstr_h, out_h, acc_sh,
          rows0, rows1, rows2, im0, im1, im2, nm0, nm1, nm2, dm0, dm1, dm2,
          zbuf, gs0, gs1, gs2, ss0, ss1, ss2):
        cid = lax.axis_index("c")
        sid = lax.axis_index("s")
        wid = sid * NC + cid
        row0 = wid * RPT
        BUF = [(rows0, im0, nm0, dm0, gs0, ss0),
               (rows1, im1, nm1, dm1, gs1, ss1),
               (rows2, im2, nm2, dm2, gs2, ss2)]

        def zinit(i, _):
            for j in range(D // L):
                zbuf[i, pl.ds(j * L, L)] = jnp.zeros((L,), jnp.float32)
            return _
        lax.fori_loop(0, 25, zinit, None)
        def zcopy(i, _):
            pltpu.sync_copy(zbuf, acc_sh.at[pl.ds(sid * 625 + i * 25, 25)])
            return _
        lax.fori_loop(0, 625 // 25, zcopy, None)
        plsc.subcore_barrier()

        def load_meta(g, B):
            _, im, nm, dm, _, _ = B
            pltpu.sync_copy(gidx_h.at[pl.ds(row0 + g * GR, GR)], im)
            pltpu.sync_copy(norm_h.at[pl.ds(row0 + g * GR, GR)], nm)
            pltpu.sync_copy(dstr_h.at[pl.ds(row0 + g * GR, GR)], dm)

        def fire_g(B):
            rows, im, _, _, gs, _ = B
            for b in range(GR):
                pltpu.async_copy(y_h.at[im.at[b]], rows.at[b], gs)

        def drain_g(B):
            rows, im, _, _, gs, _ = B
            for b in range(GR):
                pltpu.make_async_copy(y_h.at[im.at[b]], rows.at[b], gs).wait()

        def scale(B):
            rows, _, nm, _, _, _ = B
            def srow(b, _):
                rspl = jnp.full((L,), b, jnp.int32)
                def sq(q, _):
                    for t in range(L):
                        e = q * L + t
                        nspl = plsc.load_gather(
                            nm, [rspl, jnp.full((L,), e, jnp.int32)])
                        for j in range(D // L):
                            sj = pl.ds(j * L, L)
                            rows[b, e, sj] = rows[b, e, sj] * nspl
                    return _
                lax.fori_loop(0, W // L, sq, None)
                return _
            lax.fori_loop(0, GR, srow, None)

        def fire_s(B):
            rows, _, _, dm, _, ss = B
            for b in range(GR):
                pltpu.async_copy(rows.at[b], acc_sh.at[dm.at[b]], ss,
                                 add=True)

        def drain_s(B):
            rows, _, _, dm, _, ss = B
            for b in range(GR):
                pltpu.make_async_copy(rows.at[b], acc_sh.at[dm.at[b]],
                                      ss).wait()

        def step(g, bi, first_s, do_fire):
            Bg = BUF[bi]
            drain_g(Bg)
            scale(Bg)
            fire_s(Bg)
            Bn = BUF[(bi + 2) % NB]
            if not first_s:
                drain_s(Bn)
            if do_fire:
                load_meta(g + 2, Bn)
                fire_g(Bn)

        load_meta(0, BUF[0]); fire_g(BUF[0])
        load_meta(1, BUF[1]); fire_g(BUF[1])
        step(0, 0, True, True)
        step(1, 1, False, True)
        step(2, 2, False, True)

        def trip(p, _):
            g0 = 3 * p
            step(g0, 0, False, True)
            step(g0 + 1, 1, False, True)
            step(g0 + 2, 2, False, True)
            return _
        lax.fori_loop(1, (NG - 3) // 3, trip, None)
        step(21, 0, False, True)
        step(22, 1, False, True)
        step(23, 2, False, False)
        step(24, 0, False, False)
        drain_s(BUF[24 % NB])

        plsc.subcore_barrier()

        pltpu.sync_copy(acc_sh.at[pl.ds(sid * 625, 625)],
                        out_h.at[cid, pl.ds(sid * 625, 625)])

    return k(y_flat, gidx, norm, dstr)


def _tc_matmul_aug(x, w, rootv):
    _, K, D = w.shape
    BN = 1000

    def body(x_ref, w_ref, r_ref, y_ref):
        r = pl.program_id(0)
        wm = jnp.where(r == R, r_ref[0], w_ref[0])
        y_ref[0] = jnp.dot(x_ref[...], wm, preferred_element_type=jnp.float32)

    return pl.pallas_call(
        body,
        grid=(R + 1, N // BN),
        in_specs=[pl.BlockSpec((BN, K), lambda r, i: (i, 0)),
                  pl.BlockSpec((1, K, D), lambda r, i: (jnp.minimum(r, R - 1), 0, 0)),
                  pl.BlockSpec((1, K, D), lambda r, i: (0, 0, 0))],
        out_specs=pl.BlockSpec((1, BN, D), lambda r, i: (r, i, 0)),
        out_shape=jax.ShapeDtypeStruct((R + 1, N, D), jnp.float32),
    )(x, w, rootv)


def _tc_relu_matmul(aggp, y_prev, bias, w, rootv):
    _, K, D = w.shape
    BN = 1000

    def body(a_ref, u_ref, b_ref, w_ref, r_ref, y_ref):
        r = pl.program_id(0)
        h = jnp.maximum(a_ref[0] + a_ref[1] + u_ref[0] + b_ref[...], 0.0)
        wm = jnp.where(r == R, r_ref[0], w_ref[0])
        y_ref[0] = jnp.dot(h, wm, preferred_element_type=jnp.float32)

    return pl.pallas_call(
        body,
        grid=(R + 1, N // BN),
        in_specs=[pl.BlockSpec((2, BN, K), lambda r, i: (0, i, 0)),
                  pl.BlockSpec((1, BN, K), lambda r, i: (R, i, 0)),
                  pl.BlockSpec((1, K), lambda r, i: (0, 0)),
                  pl.BlockSpec((1, K, D), lambda r, i: (jnp.minimum(r, R - 1), 0, 0)),
                  pl.BlockSpec((1, K, D), lambda r, i: (0, 0, 0))],
        out_specs=pl.BlockSpec((1, BN, D), lambda r, i: (r, i, 0)),
        out_shape=jax.ShapeDtypeStruct((R + 1, N, D), jnp.float32),
    )(aggp, y_prev, bias, w, rootv)


def _tc_head(aggp, y_prev, bias, lw, lb):
    K = lw.shape[0]
    BN = 1000

    def body(a_ref, u_ref, b_ref, w_ref, lb_ref, o_ref):
        h = a_ref[0] + a_ref[1] + u_ref[0] + b_ref[...]
        o_ref[...] = jnp.dot(h, w_ref[...],
                             preferred_element_type=jnp.float32) + lb_ref[...]

    return pl.pallas_call(
        body,
        grid=(N // BN,),
        in_specs=[pl.BlockSpec((2, BN, K), lambda i: (0, i, 0)),
                  pl.BlockSpec((1, BN, K), lambda i: (R, i, 0)),
                  pl.BlockSpec((1, K), lambda i: (0, 0)),
                  pl.BlockSpec((K, 2), lambda i: (0, 0)),
                  pl.BlockSpec((1, 2), lambda i: (0, 0))],
        out_specs=pl.BlockSpec((BN, 2), lambda i: (i, 0)),
        out_shape=jax.ShapeDtypeStruct((N, 2), jnp.float32),
    )(aggp, y_prev, bias, lw, lb)


def kernel(x, edge_index, edge_type, weight1, root1, bias1,
           weight2, root2, bias2, lin_w, lin_b):
    src = edge_index[0]
    dst = edge_index[1]

    gidx, norm, dstr = _sc_counts_norm(src, dst, edge_type)
    y1 = _tc_matmul_aug(x, weight1, root1[None])
    agg1 = _sc_msgpass(y1.reshape((R + 1) * N, 32),
                       gidx, norm, dstr, 32)
    y2 = _tc_relu_matmul(agg1, y1, bias1[None], weight2, root2[None])
    agg2 = _sc_msgpass(y2.reshape((R + 1) * N, 64),
                       gidx, norm, dstr, 64)
    return _tc_head(agg2, y2, bias2[None], lin_w, lin_b[None])


# speedup vs baseline: 1.2781x; 1.2781x over previous
import functools

import jax
import jax.numpy as jnp
from jax import lax
from jax.experimental import pallas as pl
from jax.experimental.pallas import tpu as pltpu
from jax.experimental.pallas import tpu_sc as plsc

N = 10000
E = 320000
R = 8
NC, NS = 2, 16
NW = NC * NS
W = 80
NROWS = E // W
RPT = NROWS // NW
RPT_CNT = NROWS // NS
CNT_PAD = 81920
L = 16


def _vsl(j):
    return (j // 5, pl.ds((j % 5) * L, L))


def _sc_counts_norm(src, dst, etype):
    mesh = plsc.VectorSubcoreMesh(core_axis_name="c", subcore_axis_name="s")
    CH = 2000
    CR = CH // W

    @functools.partial(
        pl.kernel,
        out_type=[
            jax.ShapeDtypeStruct((NROWS, W), jnp.int32),
            jax.ShapeDtypeStruct((NROWS, W), jnp.float32),
            jax.ShapeDtypeStruct((NROWS, W), jnp.int32),
        ],
        mesh=mesh,
        compiler_params=pltpu.CompilerParams(use_tc_tiling_on_sc=False, needs_layout_passes=False),
        scratch_types=[
            pltpu.VMEM_SHARED((CNT_PAD,), jnp.float32),
            pltpu.VMEM((CH,), jnp.int32),
            pltpu.VMEM((CH,), jnp.int32),
            pltpu.VMEM((CH,), jnp.int32),
            pltpu.VMEM((CR, W), jnp.int32),
            pltpu.VMEM((CR, W), jnp.int32),
            pltpu.VMEM((CR, W), jnp.int32),
            pltpu.VMEM((CR, W), jnp.float32),
            pltpu.VMEM((W,), jnp.float32),
            pltpu.VMEM((1024,), jnp.float32),
            pltpu.VMEM((CNT_PAD,), jnp.float32),
            pltpu.SemaphoreType.DMA,
        ],
    )
    def k(src_h, dst_h, type_h, gidx_h, norm_h, dstr_h,
          counts_sh, sbuf, dbuf, tbuf, cidx, gbuf, drows, nrows_, ones, zbuf,
          cloc, csem):
        cid = lax.axis_index("c")
        sid = lax.axis_index("s")
        wid = sid * NC + cid

        def initz(i, _):
            zbuf[pl.ds(i * L, L)] = jnp.zeros((L,), jnp.float32)
            return _
        lax.fori_loop(0, 1024 // L, initz, None)
        for j in range(W // L):
            ones[pl.ds(j * L, L)] = jnp.ones((L,), jnp.float32)
        def zrow(i, _):
            pltpu.sync_copy(zbuf, counts_sh.at[pl.ds(sid * 5120 + i * 1024, 1024)])
            return _
        lax.fori_loop(0, 5, zrow, None)
        plsc.subcore_barrier()

        def count_chunk(c, _):
            base = sid * (RPT_CNT * W) + c * CH
            pltpu.sync_copy(dst_h.at[pl.ds(base, CH)], dbuf)
            pltpu.sync_copy(type_h.at[pl.ds(base, CH)], tbuf)

            def mkcidx(r, _):
                for j in range(W // L):
                    s = r * W + j * L
                    d = dbuf[pl.ds(s, L)]
                    t = tbuf[pl.ds(s, L)]
                    cidx[r, pl.ds(j * L, L)] = d * R + t
                return _
            lax.fori_loop(0, CR, mkcidx, None)

            def scat(r, _):
                pltpu.async_copy(ones, counts_sh.at[cidx.at[r]], csem,
                                 add=True)
                return _
            lax.fori_loop(0, CR, scat, None)
            def scatw(r, _):
                pltpu.make_async_copy(ones, counts_sh.at[cidx.at[r]],
                                      csem).wait()
                return _
            lax.fori_loop(0, CR, scatw, None)
            return _
        lax.fori_loop(0, RPT_CNT * W // CH, count_chunk, None)
        plsc.subcore_barrier()

        pltpu.sync_copy(counts_sh, cloc)

        def norm_chunk(c, _):
            base = wid * (RPT * W) + c * CH
            row0 = wid * RPT + c * CR
            pltpu.sync_copy(src_h.at[pl.ds(base, CH)], sbuf)
            pltpu.sync_copy(dst_h.at[pl.ds(base, CH)], dbuf)
            pltpu.sync_copy(type_h.at[pl.ds(base, CH)], tbuf)

            def mkrow(r, _):
                for j in range(W // L):
                    s = r * W + j * L
                    sj = pl.ds(j * L, L)
                    d = dbuf[pl.ds(s, L)]
                    t = tbuf[pl.ds(s, L)]
                    sc = sbuf[pl.ds(s, L)]
                    gbuf[r, sj] = t * N + sc
                    drows[r, sj] = d
                    cnt = plsc.load_gather(cloc, [d * R + t])
                    nrows_[r, sj] = 1.0 / jnp.maximum(cnt, 1.0)
                return _
            lax.fori_loop(0, CR, mkrow, None)

            pltpu.sync_copy(gbuf, gidx_h.at[pl.ds(row0, CR)])
            pltpu.sync_copy(nrows_, norm_h.at[pl.ds(row0, CR)])
            pltpu.sync_copy(drows, dstr_h.at[pl.ds(row0, CR)])
            return _
        lax.fori_loop(0, RPT * W // CH, norm_chunk, None)

    return k(src, dst, etype)


def _sc_msgpass(y_flat, gidx, norm, dstr, D):
    mesh = plsc.VectorSubcoreMesh(core_axis_name="c", subcore_axis_name="s")
    CR = 5
    NCH = RPT // CR
    ZR = 125

    @functools.partial(
        pl.kernel,
        out_type=jax.ShapeDtypeStruct((NC, N, D), jnp.float32),
        mesh=mesh,
        compiler_params=pltpu.CompilerParams(use_tc_tiling_on_sc=False, needs_layout_passes=False),
        scratch_types=[
            pltpu.VMEM_SHARED((N, D), jnp.float32),
            pltpu.VMEM((RPT, W), jnp.int32),
            pltpu.VMEM((RPT, W), jnp.float32),
            pltpu.VMEM((RPT, W), jnp.int32),
            pltpu.VMEM((CR, W, D), jnp.float32),
            pltpu.VMEM((CR, W, D), jnp.float32),
            pltpu.VMEM((ZR, D), jnp.float32),
            pltpu.SemaphoreType.DMA,
            pltpu.SemaphoreType.DMA,
        ],
    )
    def k(y_h, gidx_h, norm_h, dstr_h, out_h, acc_sh, ibuf, nbuf, dbuf,
          rows0, rows1, zbuf, sem0, sem1):
        cid = lax.axis_index("c")
        sid = lax.axis_index("s")
        wid = sid * NC + cid
        row0 = wid * RPT

        def zinit(i, _):
            for j in range(D // L):
                zbuf[i, pl.ds(j * L, L)] = jnp.zeros((L,), jnp.float32)
            return _
        lax.fori_loop(0, ZR, zinit, None)
        def zcopy(i, _):
            pltpu.sync_copy(zbuf, acc_sh.at[pl.ds(sid * 625 + i * ZR, ZR)])
            return _
        lax.fori_loop(0, 625 // ZR, zcopy, None)

        pltpu.sync_copy(gidx_h.at[pl.ds(row0, RPT)], ibuf)
        pltpu.sync_copy(norm_h.at[pl.ds(row0, RPT)], nbuf)
        pltpu.sync_copy(dstr_h.at[pl.ds(row0, RPT)], dbuf)
        plsc.subcore_barrier()

        def fire(g, rows, sem):
            for b in range(CR):
                pltpu.async_copy(y_h.at[ibuf.at[g * CR + b]], rows.at[b], sem)

        def drain(g, rows, sem):
            for b in range(CR):
                pltpu.make_async_copy(y_h.at[ibuf.at[g * CR + b]],
                                      rows.at[b], sem).wait()

        def scale(g, rows):
            def srow(r, _):
                rspl = jnp.full((L,), g * CR + r, jnp.int32)
                for q in range(W // L):
                    for t in range(L):
                        e = q * L + t
                        nspl = plsc.load_gather(
                            nbuf, [rspl, jnp.full((L,), e, jnp.int32)])
                        for j in range(D // L):
                            sj = pl.ds(j * L, L)
                            rows[r, e, sj] = rows[r, e, sj] * nspl
                return _
            lax.fori_loop(0, CR, srow, None)

        def scat(g, rows):
            def sc1(b, _):
                pltpu.sync_copy(rows.at[b], acc_sh.at[dbuf.at[g * CR + b]],
                                add=True)
                return _
            lax.fori_loop(0, CR, sc1, None)

        fire(0, rows0, sem0)

        def pair(p, _):
            g0 = 2 * p
            fire(g0 + 1, rows1, sem1)
            drain(g0, rows0, sem0)
            scale(g0, rows0)
            scat(g0, rows0)
            fire(g0 + 2, rows0, sem0)
            drain(g0 + 1, rows1, sem1)
            scale(g0 + 1, rows1)
            scat(g0 + 1, rows1)
            return _
        lax.fori_loop(0, (NCH - 1) // 2, pair, None)
        drain(NCH - 1, rows0, sem0)
        scale(NCH - 1, rows0)
        scat(NCH - 1, rows0)

        plsc.subcore_barrier()

        pltpu.sync_copy(acc_sh.at[pl.ds(sid * 625, 625)],
                        out_h.at[cid, pl.ds(sid * 625, 625)])

    return k(y_flat, gidx, norm, dstr)


def _tc_matmul_aug(x, w, rootv):
    _, K, D = w.shape
    BN = 1000

    def body(x_ref, w_ref, r_ref, y_ref):
        r = pl.program_id(0)
        wm = jnp.where(r == R, r_ref[0], w_ref[0])
        y_ref[0] = jnp.dot(x_ref[...], wm, preferred_element_type=jnp.float32)

    return pl.pallas_call(
        body,
        grid=(R + 1, N // BN),
        in_specs=[pl.BlockSpec((BN, K), lambda r, i: (i, 0)),
                  pl.BlockSpec((1, K, D), lambda r, i: (jnp.minimum(r, R - 1), 0, 0)),
                  pl.BlockSpec((1, K, D), lambda r, i: (0, 0, 0))],
        out_specs=pl.BlockSpec((1, BN, D), lambda r, i: (r, i, 0)),
        out_shape=jax.ShapeDtypeStruct((R + 1, N, D), jnp.float32),
    )(x, w, rootv)


def _tc_relu_matmul(aggp, y_prev, bias, w, rootv):
    _, K, D = w.shape
    BN = 1000

    def body(a_ref, u_ref, b_ref, w_ref, r_ref, y_ref):
        r = pl.program_id(0)
        h = jnp.maximum(a_ref[0] + a_ref[1] + u_ref[0] + b_ref[...], 0.0)
        wm = jnp.where(r == R, r_ref[0], w_ref[0])
        y_ref[0] = jnp.dot(h, wm, preferred_element_type=jnp.float32)

    return pl.pallas_call(
        body,
        grid=(R + 1, N // BN),
        in_specs=[pl.BlockSpec((2, BN, K), lambda r, i: (0, i, 0)),
                  pl.BlockSpec((1, BN, K), lambda r, i: (R, i, 0)),
                  pl.BlockSpec((1, K), lambda r, i: (0, 0)),
                  pl.BlockSpec((1, K, D), lambda r, i: (jnp.minimum(r, R - 1), 0, 0)),
                  pl.BlockSpec((1, K, D), lambda r, i: (0, 0, 0))],
        out_specs=pl.BlockSpec((1, BN, D), lambda r, i: (r, i, 0)),
        out_shape=jax.ShapeDtypeStruct((R + 1, N, D), jnp.float32),
    )(aggp, y_prev, bias, w, rootv)


def _tc_head(aggp, y_prev, bias, lw, lb):
    K = lw.shape[0]
    BN = 1000

    def body(a_ref, u_ref, b_ref, w_ref, lb_ref, o_ref):
        h = a_ref[0] + a_ref[1] + u_ref[0] + b_ref[...]
        o_ref[...] = jnp.dot(h, w_ref[...],
                             preferred_element_type=jnp.float32) + lb_ref[...]

    return pl.pallas_call(
        body,
        grid=(N // BN,),
        in_specs=[pl.BlockSpec((2, BN, K), lambda i: (0, i, 0)),
                  pl.BlockSpec((1, BN, K), lambda i: (R, i, 0)),
                  pl.BlockSpec((1, K), lambda i: (0, 0)),
                  pl.BlockSpec((K, 2), lambda i: (0, 0)),
                  pl.BlockSpec((1, 2), lambda i: (0, 0))],
        out_specs=pl.BlockSpec((BN, 2), lambda i: (i, 0)),
        out_shape=jax.ShapeDtypeStruct((N, 2), jnp.float32),
    )(aggp, y_prev, bias, lw, lb)


def kernel(x, edge_index, edge_type, weight1, root1, bias1,
           weight2, root2, bias2, lin_w, lin_b):
    src = edge_index[0]
    dst = edge_index[1]

    gidx, norm, dstr = _sc_counts_norm(src, dst, edge_type)
    y1 = _tc_matmul_aug(x, weight1, root1[None])
    agg1 = _sc_msgpass(y1.reshape((R + 1) * N, 32),
                       gidx, norm, dstr, 32)
    y2 = _tc_relu_matmul(agg1, y1, bias1[None], weight2, root2[None])
    agg2 = _sc_msgpass(y2.reshape((R + 1) * N, 64),
                       gidx, norm, dstr, 64)
    return _tc_head(agg2, y2, bias2[None], lin_w, lin_b[None])


# speedup vs baseline: 1.5135x; 1.1842x over previous
import functools

import jax
import jax.numpy as jnp
from jax import lax
from jax.experimental import pallas as pl
from jax.experimental.pallas import tpu as pltpu
from jax.experimental.pallas import tpu_sc as plsc

N = 10000
E = 320000
R = 8
NC, NS = 2, 16
NW = NC * NS
W = 80
NROWS = E // W
RPT = NROWS // NW
RPT_CNT = NROWS // NS
CNT_PAD = 81920
L = 16


def _sc_counts_norm(src, dst, etype):
    mesh = plsc.VectorSubcoreMesh(core_axis_name="c", subcore_axis_name="s")
    CH = 2000
    CR = CH // W

    @functools.partial(
        pl.kernel,
        out_type=[
            jax.ShapeDtypeStruct((NROWS, W), jnp.int32),
            jax.ShapeDtypeStruct((NROWS, W), jnp.float32),
            jax.ShapeDtypeStruct((NROWS, W), jnp.int32),
        ],
        mesh=mesh,
        compiler_params=pltpu.CompilerParams(use_tc_tiling_on_sc=False, needs_layout_passes=False),
        scratch_types=[
            pltpu.VMEM_SHARED((CNT_PAD,), jnp.float32),
            pltpu.VMEM((CH,), jnp.int32),
            pltpu.VMEM((CH,), jnp.int32),
            pltpu.VMEM((CH,), jnp.int32),
            pltpu.VMEM((CR, W), jnp.int32),
            pltpu.VMEM((CR, W), jnp.int32),
            pltpu.VMEM((CR, W), jnp.int32),
            pltpu.VMEM((CR, W), jnp.float32),
            pltpu.VMEM((W,), jnp.float32),
            pltpu.VMEM((1024,), jnp.float32),
            pltpu.VMEM((CNT_PAD,), jnp.float32),
            pltpu.SemaphoreType.DMA,
        ],
    )
    def k(src_h, dst_h, type_h, gidx_h, norm_h, dstr_h,
          counts_sh, sbuf, dbuf, tbuf, cidx, gbuf, drows, nrows_, ones, zbuf,
          cloc, csem):
        cid = lax.axis_index("c")
        sid = lax.axis_index("s")
        wid = sid * NC + cid

        def initz(i, _):
            zbuf[pl.ds(i * L, L)] = jnp.zeros((L,), jnp.float32)
            return _
        lax.fori_loop(0, 1024 // L, initz, None)
        for j in range(W // L):
            ones[pl.ds(j * L, L)] = jnp.ones((L,), jnp.float32)
        def zrow(i, _):
            pltpu.sync_copy(zbuf, counts_sh.at[pl.ds(sid * 5120 + i * 1024, 1024)])
            return _
        lax.fori_loop(0, 5, zrow, None)
        plsc.subcore_barrier()

        def count_chunk(c, _):
            base = sid * (RPT_CNT * W) + c * CH
            pltpu.sync_copy(dst_h.at[pl.ds(base, CH)], dbuf)
            pltpu.sync_copy(type_h.at[pl.ds(base, CH)], tbuf)

            def mkcidx(r, _):
                for j in range(W // L):
                    s = r * W + j * L
                    d = dbuf[pl.ds(s, L)]
                    t = tbuf[pl.ds(s, L)]
                    cidx[r, pl.ds(j * L, L)] = d * R + t
                return _
            lax.fori_loop(0, CR, mkcidx, None)

            def scat(r, _):
                pltpu.async_copy(ones, counts_sh.at[cidx.at[r]], csem,
                                 add=True)
                return _
            lax.fori_loop(0, CR, scat, None)
            def scatw(r, _):
                pltpu.make_async_copy(ones, counts_sh.at[cidx.at[r]],
                                      csem).wait()
                return _
            lax.fori_loop(0, CR, scatw, None)
            return _
        lax.fori_loop(0, RPT_CNT * W // CH, count_chunk, None)
        plsc.subcore_barrier()

        pltpu.sync_copy(counts_sh, cloc)

        def norm_chunk(c, _):
            base = wid * (RPT * W) + c * CH
            row0 = wid * RPT + c * CR
            pltpu.sync_copy(src_h.at[pl.ds(base, CH)], sbuf)
            pltpu.sync_copy(dst_h.at[pl.ds(base, CH)], dbuf)
            pltpu.sync_copy(type_h.at[pl.ds(base, CH)], tbuf)

            def mkrow(r, _):
                for j in range(W // L):
                    s = r * W + j * L
                    sj = pl.ds(j * L, L)
                    d = dbuf[pl.ds(s, L)]
                    t = tbuf[pl.ds(s, L)]
                    sc = sbuf[pl.ds(s, L)]
                    gbuf[r, sj] = t * N + sc
                    drows[r, sj] = d
                    cnt = plsc.load_gather(cloc, [d * R + t])
                    nrows_[r, sj] = 1.0 / jnp.maximum(cnt, 1.0)
                return _
            lax.fori_loop(0, CR, mkrow, None)

            pltpu.sync_copy(gbuf, gidx_h.at[pl.ds(row0, CR)])
            pltpu.sync_copy(nrows_, norm_h.at[pl.ds(row0, CR)])
            pltpu.sync_copy(drows, dstr_h.at[pl.ds(row0, CR)])
            return _
        lax.fori_loop(0, RPT * W // CH, norm_chunk, None)

    return k(src, dst, etype)


def _sc_msgpass(y_flat, gidx, norm, dstr, D):
    mesh = plsc.VectorSubcoreMesh(core_axis_name="c", subcore_axis_name="s")
    CR = 5
    NCH = RPT // CR
    ZR = 125

    @functools.partial(
        pl.kernel,
        out_type=jax.ShapeDtypeStruct((NC, N, D), jnp.float32),
        mesh=mesh,
        compiler_params=pltpu.CompilerParams(use_tc_tiling_on_sc=False, needs_layout_passes=False),
        scratch_types=[
            pltpu.VMEM_SHARED((N, D), jnp.float32),
            pltpu.VMEM((RPT, W), jnp.int32),
            pltpu.VMEM((RPT, W), jnp.float32),
            pltpu.VMEM((RPT, W), jnp.int32),
            pltpu.VMEM((CR, W, D), jnp.float32),
            pltpu.VMEM((CR, W, D), jnp.float32),
            pltpu.VMEM((ZR, D), jnp.float32),
            pltpu.SemaphoreType.DMA,
            pltpu.SemaphoreType.DMA,
        ],
    )
    def k(y_h, gidx_h, norm_h, dstr_h, out_h, acc_sh, ibuf, nbuf, dbuf,
          rows0, rows1, zbuf, sem0, sem1):
        cid = lax.axis_index("c")
        sid = lax.axis_index("s")
        wid = sid * NC + cid
        row0 = wid * RPT

        def zinit(i, _):
            for j in range(D // L):
                zbuf[i, pl.ds(j * L, L)] = jnp.zeros((L,), jnp.float32)
            return _
        lax.fori_loop(0, ZR, zinit, None)
        def zcopy(i, _):
            pltpu.sync_copy(zbuf, acc_sh.at[pl.ds(sid * 625 + i * ZR, ZR)])
            return _
        lax.fori_loop(0, 625 // ZR, zcopy, None)

        pltpu.sync_copy(gidx_h.at[pl.ds(row0, RPT)], ibuf)
        pltpu.sync_copy(norm_h.at[pl.ds(row0, RPT)], nbuf)
        pltpu.sync_copy(dstr_h.at[pl.ds(row0, RPT)], dbuf)
        plsc.subcore_barrier()

        def fire(g, rows, sem):
            for b in range(CR):
                pltpu.async_copy(y_h.at[ibuf.at[g * CR + b]], rows.at[b], sem)

        def drain(g, rows, sem):
            for b in range(CR):
                pltpu.make_async_copy(y_h.at[ibuf.at[g * CR + b]],
                                      rows.at[b], sem).wait()

        _dn = lax.GatherDimensionNumbers(offset_dims=(),
                                         collapsed_slice_dims=(0,),
                                         start_index_map=(0,))

        def scale(g, rows):
            def srow(r, _):
                for q in range(W // L):
                    n16 = nbuf[g * CR + r, pl.ds(q * L, L)]
                    for t in range(L):
                        e = q * L + t
                        nspl = lax.gather(
                            n16, jnp.full((L, 1), t, jnp.int32), _dn, (1,),
                            mode=lax.GatherScatterMode.PROMISE_IN_BOUNDS)
                        for j in range(D // L):
                            sj = pl.ds(j * L, L)
                            rows[r, e, sj] = rows[r, e, sj] * nspl
                return _
            lax.fori_loop(0, CR, srow, None)

        def scat(g, rows):
            def sc1(b, _):
                pltpu.sync_copy(rows.at[b], acc_sh.at[dbuf.at[g * CR + b]],
                                add=True)
                return _
            lax.fori_loop(0, CR, sc1, None)

        fire(0, rows0, sem0)

        def pair(p, _):
            g0 = 2 * p
            fire(g0 + 1, rows1, sem1)
            drain(g0, rows0, sem0)
            scale(g0, rows0)
            scat(g0, rows0)
            fire(g0 + 2, rows0, sem0)
            drain(g0 + 1, rows1, sem1)
            scale(g0 + 1, rows1)
            scat(g0 + 1, rows1)
            return _
        lax.fori_loop(0, (NCH - 1) // 2, pair, None)
        drain(NCH - 1, rows0, sem0)
        scale(NCH - 1, rows0)
        scat(NCH - 1, rows0)

        plsc.subcore_barrier()

        pltpu.sync_copy(acc_sh.at[pl.ds(sid * 625, 625)],
                        out_h.at[cid, pl.ds(sid * 625, 625)])

    return k(y_flat, gidx, norm, dstr)


def _tc_matmul_aug(x, w, rootv):
    _, K, D = w.shape
    BN = 1000

    def body(x_ref, w_ref, r_ref, y_ref):
        r = pl.program_id(0)
        wm = jnp.where(r == R, r_ref[0], w_ref[0])
        y_ref[0] = jnp.dot(x_ref[...], wm, preferred_element_type=jnp.float32)

    return pl.pallas_call(
        body,
        grid=(R + 1, N // BN),
        in_specs=[pl.BlockSpec((BN, K), lambda r, i: (i, 0)),
                  pl.BlockSpec((1, K, D), lambda r, i: (jnp.minimum(r, R - 1), 0, 0)),
                  pl.BlockSpec((1, K, D), lambda r, i: (0, 0, 0))],
        out_specs=pl.BlockSpec((1, BN, D), lambda r, i: (r, i, 0)),
        out_shape=jax.ShapeDtypeStruct((R + 1, N, D), jnp.float32),
    )(x, w, rootv)


def _tc_relu_matmul(aggp, y_prev, bias, w, rootv):
    _, K, D = w.shape
    BN = 1000

    def body(a_ref, u_ref, b_ref, w_ref, r_ref, y_ref):
        r = pl.program_id(0)
        h = jnp.maximum(a_ref[0] + a_ref[1] + u_ref[0] + b_ref[...], 0.0)
        wm = jnp.where(r == R, r_ref[0], w_ref[0])
        y_ref[0] = jnp.dot(h, wm, preferred_element_type=jnp.float32)

    return pl.pallas_call(
        body,
        grid=(R + 1, N // BN),
        in_specs=[pl.BlockSpec((2, BN, K), lambda r, i: (0, i, 0)),
                  pl.BlockSpec((1, BN, K), lambda r, i: (R, i, 0)),
                  pl.BlockSpec((1, K), lambda r, i: (0, 0)),
                  pl.BlockSpec((1, K, D), lambda r, i: (jnp.minimum(r, R - 1), 0, 0)),
                  pl.BlockSpec((1, K, D), lambda r, i: (0, 0, 0))],
        out_specs=pl.BlockSpec((1, BN, D), lambda r, i: (r, i, 0)),
        out_shape=jax.ShapeDtypeStruct((R + 1, N, D), jnp.float32),
    )(aggp, y_prev, bias, w, rootv)


def _tc_head(aggp, y_prev, bias, lw, lb):
    K = lw.shape[0]
    BN = 1000

    def body(a_ref, u_ref, b_ref, w_ref, lb_ref, o_ref):
        h = a_ref[0] + a_ref[1] + u_ref[0] + b_ref[...]
        o_ref[...] = jnp.dot(h, w_ref[...],
                             preferred_element_type=jnp.float32) + lb_ref[...]

    return pl.pallas_call(
        body,
        grid=(N // BN,),
        in_specs=[pl.BlockSpec((2, BN, K), lambda i: (0, i, 0)),
                  pl.BlockSpec((1, BN, K), lambda i: (R, i, 0)),
                  pl.BlockSpec((1, K), lambda i: (0, 0)),
                  pl.BlockSpec((K, 2), lambda i: (0, 0)),
                  pl.BlockSpec((1, 2), lambda i: (0, 0))],
        out_specs=pl.BlockSpec((BN, 2), lambda i: (i, 0)),
        out_shape=jax.ShapeDtypeStruct((N, 2), jnp.float32),
    )(aggp, y_prev, bias, lw, lb)


def kernel(x, edge_index, edge_type, weight1, root1, bias1,
           weight2, root2, bias2, lin_w, lin_b):
    src = edge_index[0]
    dst = edge_index[1]

    gidx, norm, dstr = _sc_counts_norm(src, dst, edge_type)
    y1 = _tc_matmul_aug(x, weight1, root1[None])
    agg1 = _sc_msgpass(y1.reshape((R + 1) * N, 32),
                       gidx, norm, dstr, 32)
    y2 = _tc_relu_matmul(agg1, y1, bias1[None], weight2, root2[None])
    agg2 = _sc_msgpass(y2.reshape((R + 1) * N, 64),
                       gidx, norm, dstr, 64)
    return _tc_head(agg2, y2, bias2[None], lin_w, lin_b[None])
